# Initial kernel scaffold; baseline (speedup 1.0000x reference)
import jax
import jax.numpy as jnp
from jax.experimental import pallas as pl


def kernel(x, edge_index, W_proj, b_proj, g_proj, be_proj, W1, b1, g1, be1, W2, b2, g2, be2, W3, b3, g3, be3):
    raise NotImplementedError("write your pallas kernel here")


# speedup vs baseline: 8.2182x; 8.2182x over previous
import functools

import jax
import jax.numpy as jnp
from jax import lax
from jax.experimental import pallas as pl
from jax.experimental.pallas import tpu as pltpu
from jax.experimental.pallas import tpu_sc as plsc

N = 10000
E = 320000
D = 128

NC = 2
NS = 16
NW = NC * NS
CHUNK = 128
EPW = 10112
NCHUNK = EPW // CHUNK
EPAD = NW * EPW
NP = 10112
RPT = NP // NS

@functools.cache
def _get_mesh():
  return plsc.VectorSubcoreMesh(core_axis_name="c", subcore_axis_name="s",
                                num_cores=NC, num_subcores=NS)


def _sc_scatter_body(s_hbm, rows_hbm, cols_hbm, zeros_hbm, out_hbm,
                     rowbuf, colbuf, gbuf, sem, acc):
  c = lax.axis_index("c")
  s = lax.axis_index("s")
  wid = c * NS + s
  pltpu.sync_copy(rows_hbm.at[wid], rowbuf)
  pltpu.sync_copy(cols_hbm.at[wid], colbuf)
  last = s == NS - 1

  @pl.when(jnp.logical_and(c == 0, jnp.logical_not(last)))
  def _():
    pltpu.sync_copy(s_hbm.at[pl.ds(s * RPT, RPT)], acc.at[pl.ds(s * RPT, RPT)])

  @pl.when(jnp.logical_and(c == 0, last))
  def _():
    pltpu.sync_copy(s_hbm.at[pl.ds((NS - 1) * RPT, N - (NS - 1) * RPT)],
                    acc.at[pl.ds((NS - 1) * RPT, N - (NS - 1) * RPT)])

  @pl.when(c == 1)
  def _():
    pltpu.sync_copy(zeros_hbm.at[pl.ds(s * RPT, RPT)],
                    acc.at[pl.ds(s * RPT, RPT)])

  plsc.subcore_barrier()

  def body(j, carry):
    pltpu.async_copy(s_hbm.at[rowbuf.at[j]], gbuf, sem).wait()
    pltpu.sync_copy(gbuf, acc.at[colbuf.at[j]], add=True)
    return carry

  lax.fori_loop(0, NCHUNK, body, 0)
  plsc.subcore_barrier()
  pltpu.sync_copy(acc.at[pl.ds(s * RPT, RPT)], out_hbm.at[c, pl.ds(s * RPT, RPT)])


def _sc_scatter(s_val, rows3d, cols3d, zerosN):
  k = pl.kernel(
      _sc_scatter_body,
      out_type=jax.ShapeDtypeStruct((NC, NP, D), jnp.float32),
      mesh=_get_mesh(),
      scratch_types=[
          pltpu.VMEM((NCHUNK, CHUNK), jnp.int32),
          pltpu.VMEM((NCHUNK, CHUNK), jnp.int32),
          pltpu.VMEM((CHUNK, D), jnp.float32),
          pltpu.SemaphoreType.DMA,
          pltpu.VMEM_SHARED((NP, D), jnp.float32),
      ],
  )
  return k(s_val, rows3d, cols3d, zerosN)


R = 1000
G = N // R

_DOT = dict(precision=lax.Precision.HIGHEST, preferred_element_type=jnp.float32)


def _ln(h, g, b):
  mu = jnp.mean(h, axis=-1, keepdims=True)
  d = h - mu
  var = jnp.mean(d * d, axis=-1, keepdims=True)
  return d * lax.rsqrt(var + 1e-5) * g + b


def _dinv(dd):
  return lax.rsqrt(dd[0, :, 0:1] + dd[1, :, 0:1])


def _gelu(x):
  return 0.5 * x * (1.0 + lax.erf(x * 0.7071067811865476))


def _tc1_body(x_ref, wp_ref, bp_ref, gp_ref, bep_ref, w1_ref, dd_ref,
              h0_ref, s1_ref):
  h = lax.dot_general(x_ref[...], wp_ref[...], (((1,), (1,)), ((), ())), **_DOT)
  h = _ln(h + bp_ref[...], gp_ref[...], bep_ref[...])
  h0_ref[...] = h
  s1_ref[...] = _dinv(dd_ref) * lax.dot_general(
      h, w1_ref[...], (((1,), (1,)), ((), ())), **_DOT)


def _tc_mid_body(h_ref, p_ref, dd_ref, b_ref, g_ref, be_ref, wn_ref,
                 hn_ref, sn_ref):
  dinv = _dinv(dd_ref)
  m = _gelu(dinv * (p_ref[0] + p_ref[1]) + b_ref[...])
  hn = _ln(h_ref[...] + m, g_ref[...], be_ref[...])
  hn_ref[...] = hn
  sn_ref[...] = dinv * lax.dot_general(
      hn, wn_ref[...], (((1,), (1,)), ((), ())), **_DOT)


def _tc_fin_body(h_ref, p_ref, dd_ref, b_ref, g_ref, be_ref, hn_ref):
  dinv = _dinv(dd_ref)
  m = _gelu(dinv * (p_ref[0] + p_ref[1]) + b_ref[...])
  hn_ref[...] = _ln(h_ref[...] + m, g_ref[...], be_ref[...])


_ROW = pl.BlockSpec((R, D), lambda i: (i, 0))
_W = pl.BlockSpec((D, D), lambda i: (0, 0))
_VEC = pl.BlockSpec((1, D), lambda i: (0, 0))
_DD = pl.BlockSpec((NC, R, D), lambda i: (0, i, 0))
_P = pl.BlockSpec((NC, R, D), lambda i: (0, i, 0))
_OUT2 = [jax.ShapeDtypeStruct((N, D), jnp.float32)] * 2
_OUT1 = jax.ShapeDtypeStruct((N, D), jnp.float32)


def _tc1(x, wp, bp, gp, bep, w1, dd):
  return pl.pallas_call(
      _tc1_body, grid=(G,),
      in_specs=[_ROW, _W, _VEC, _VEC, _VEC, _W, _DD],
      out_specs=[_ROW, _ROW], out_shape=_OUT2,
  )(x, wp, bp, gp, bep, w1, dd)


def _tc_mid(h, p, dd, b, g, be, wn):
  return pl.pallas_call(
      _tc_mid_body, grid=(G,),
      in_specs=[_ROW, _P, _DD, _VEC, _VEC, _VEC, _W],
      out_specs=[_ROW, _ROW], out_shape=_OUT2,
  )(h, p, dd, b, g, be, wn)


def _tc_fin(h, p, dd, b, g, be):
  return pl.pallas_call(
      _tc_fin_body, grid=(G,),
      in_specs=[_ROW, _P, _DD, _VEC, _VEC, _VEC],
      out_specs=_ROW, out_shape=_OUT1,
  )(h, p, dd, b, g, be)


def kernel(x, edge_index, W_proj, b_proj, g_proj, be_proj,
           W1, b1, g1, be1, W2, b2, g2, be2, W3, b3, g3, be3):
  pad = EPAD - E
  rows = jnp.concatenate([edge_index[0], jnp.zeros((pad,), jnp.int32)])
  cols = jnp.concatenate([edge_index[1], jnp.full((pad,), N, jnp.int32)])
  rows3d = rows.reshape(NW, NCHUNK, CHUNK)
  cols3d = cols.reshape(NW, NCHUNK, CHUNK)
  zerosN = jnp.zeros((NP, D), jnp.float32)
  onesN = jnp.ones((N, D), jnp.float32)

  dd = _sc_scatter(onesN, rows3d, cols3d, zerosN)

  r2 = lambda v: v.reshape(1, D)
  h0, s1 = _tc1(x, W_proj, r2(b_proj), r2(g_proj), r2(be_proj), W1, dd)
  p1 = _sc_scatter(s1, rows3d, cols3d, zerosN)
  h1, s2 = _tc_mid(h0, p1, dd, r2(b1), r2(g1), r2(be1), W2)
  p2 = _sc_scatter(s2, rows3d, cols3d, zerosN)
  h2, s3 = _tc_mid(h1, p2, dd, r2(b2), r2(g2), r2(be2), W3)
  p3 = _sc_scatter(s3, rows3d, cols3d, zerosN)
  return _tc_fin(h2, p3, dd, r2(b3), r2(g3), r2(be3))
